# baseline (speedup 1.0000x reference)
import functools

import jax
import jax.numpy as jnp
import numpy as np
from jax import lax
from jax.experimental import pallas as pl
from jax.experimental.pallas import tpu as pltpu
from jax.experimental.pallas import tpu_sc as plsc

_D = 32
_W = 64
_OW = 63
_NB = 16
_NR = 16
_ITEMS = _NB * _NR
_ITEMS_PER_WORKER = _ITEMS // 32
_TOK = 8 * _W
_PLANE = 4 * 4 * 8 * 128
_SCALE = np.float32(np.sqrt(32.0))


def _pos_encoding_32() -> np.ndarray:
    depth = _D / 2
    positions = np.arange(_D)[:, np.newaxis]
    depths = np.arange(depth)[np.newaxis, :] / depth
    angle_rates = 1 / 10000**depths
    angle_rads = positions * angle_rates
    return np.concatenate(
        [np.sin(angle_rads), np.cos(angle_rads)], axis=-1
    ).astype(np.float32)


_PE = _pos_encoding_32()


def _sc_body(x_hbm, pe_hbm, table_hbm, out_hbm, xv, gv, ov, pev, gsem, osem):
    wid = lax.axis_index("s") * 2 + lax.axis_index("c")
    pltpu.sync_copy(pe_hbm, pev)

    iota = lax.iota(jnp.int32, 16)
    dlo = ((iota >> 3) << 12) + ((iota & 7) << 7)
    dhi = (((iota >> 3) + 2) << 12) + ((iota & 7) << 7)

    def stage(t, slot):
        item = wid * _ITEMS_PER_WORKER + t
        b = item // _NR
        r = item % _NR
        for i in range(8):
            src = (b * 128 + i * 16 + r) * _W
            pltpu.sync_copy(
                x_hbm.at[pl.ds(src, _W)], xv.at[slot, pl.ds(i * _W, _W)]
            )
        descs = []
        for ch in range(4):
            sl = pl.ds(ch * 128, 128)
            descs.append(
                pltpu.make_async_copy(
                    table_hbm.at[xv.at[slot, sl]], gv.at[slot, sl], gsem
                )
            )
        for d in descs:
            d.start()
        return descs

    def expand(t, slot):
        item = wid * _ITEMS_PER_WORKER + t
        r = item % _NR
        pk0a = pev[2 * r, 0:16]
        pk0b = pev[2 * r, 16:32]
        pk1a = pev[2 * r + 1, 0:16]
        pk1b = pev[2 * r + 1, 16:32]
        ovf = ov.at[slot]

        def do_token(i, c, q0, q1):
            row = i * _W + c
            g0 = gv[slot, row, 0:16] * _SCALE
            g1 = gv[slot, row, 16:32] * _SCALE
            if q0:
                p = i * _OW + c
                off = ((p >> 7) << 10) + (p & 127)
                plsc.store_scatter(ovf, [dlo + off], g0 + pk0a)
                plsc.store_scatter(ovf, [dhi + off], g1 + pk0b)
            if q1:
                p1 = i * _OW + c - 1
                off1 = _PLANE + ((p1 >> 7) << 10) + (p1 & 127)
                plsc.store_scatter(ovf, [dlo + off1], g0 + pk1a)
                plsc.store_scatter(ovf, [dhi + off1], g1 + pk1b)

        for i in range(8):
            do_token(i, 0, True, False)

            @plsc.parallel_loop(1, _OW, unroll=2)
            def body(c, i=i):
                do_token(i, c, True, True)

            do_token(i, _OW, False, True)

    out_descs = [None, None]
    gdescs = stage(0, 0)
    for t in range(_ITEMS_PER_WORKER):
        slot = t % 2
        for d in gdescs:
            d.wait()
        if t + 1 < _ITEMS_PER_WORKER:
            gdescs = stage(t + 1, 1 - slot)
        if out_descs[slot] is not None:
            out_descs[slot].wait()
        expand(t, slot)
        item = wid * _ITEMS_PER_WORKER + t
        out_descs[slot] = pltpu.make_async_copy(
            ov.at[slot], out_hbm.at[pl.ds(item * 2 * _PLANE, 2 * _PLANE)], osem
        )
        out_descs[slot].start()
    for d in out_descs:
        if d is not None:
            d.wait()


@functools.partial(
    pl.kernel,
    out_type=jax.ShapeDtypeStruct((_ITEMS * 2 * _PLANE,), jnp.float32),
    mesh=plsc.VectorSubcoreMesh(core_axis_name="c", subcore_axis_name="s"),
    scratch_types=[
        pltpu.VMEM((2, _TOK), jnp.int32),
        pltpu.VMEM((2, _TOK, _D), jnp.float32),
        pltpu.VMEM((2, 2 * _PLANE), jnp.float32),
        pltpu.VMEM((_D, _D), jnp.float32),
        pltpu.SemaphoreType.DMA,
        pltpu.SemaphoreType.DMA,
    ],
    compiler_params=pltpu.CompilerParams(
        use_tc_tiling_on_sc=False, needs_layout_passes=False
    ),
)
def _pos_emb_sc(x_hbm, pe_hbm, table_hbm, out_hbm, xv, gv, ov, pev, gsem, osem):
    _sc_body(x_hbm, pe_hbm, table_hbm, out_hbm, xv, gv, ov, pev, gsem, osem)


def kernel(x, table):
    x_flat = x.reshape(-1)
    out1d = _pos_emb_sc(x_flat, _PE, table)
    out6 = out1d.reshape(_NB, _D, 4, 4, 8, 128)
    t = jnp.transpose(out6, (0, 3, 5, 1, 2, 4))
    return t.reshape(_NB, 512, _D, _D)[:, : 8 * _OW]


# speedup vs baseline: 1.1004x; 1.1004x over previous
import functools

import jax
import jax.numpy as jnp
import numpy as np
from jax import lax
from jax.experimental import pallas as pl
from jax.experimental.pallas import tpu as pltpu
from jax.experimental.pallas import tpu_sc as plsc

_D = 32
_W = 64
_OW = 63
_NB = 16
_NR = 16
_ITEMS = _NB * _NR
_ITEMS_PER_WORKER = _ITEMS // 32
_TOK = 8 * _W
_PLANE = 4 * 4 * 8 * 128
_SCALE = np.float32(np.sqrt(32.0))


def _pos_encoding_32() -> np.ndarray:
    depth = _D / 2
    positions = np.arange(_D)[:, np.newaxis]
    depths = np.arange(depth)[np.newaxis, :] / depth
    angle_rates = 1 / 10000**depths
    angle_rads = positions * angle_rates
    return np.concatenate(
        [np.sin(angle_rads), np.cos(angle_rads)], axis=-1
    ).astype(np.float32)


_PE = _pos_encoding_32()


def _sc_body(x_hbm, pe_hbm, table_hbm, out_hbm, xv, gv, ov, pev, gsem, osem):
    wid = lax.axis_index("s") * 2 + lax.axis_index("c")
    pltpu.sync_copy(pe_hbm, pev)

    iota = lax.iota(jnp.int32, 16)
    dlo = ((iota >> 3) << 12) + ((iota & 7) << 7)
    dhi = (((iota >> 3) + 2) << 12) + ((iota & 7) << 7)

    def stage(t, slot):
        item = wid * _ITEMS_PER_WORKER + t
        b = item // _NR
        r = item % _NR
        pltpu.sync_copy(x_hbm.at[b, :, r], xv.at[slot])
        descs = [
            pltpu.make_async_copy(
                table_hbm.at[xv.at[slot, i]], gv.at[slot, i], gsem
            )
            for i in range(8)
        ]
        for d in descs:
            d.start()
        return descs

    def expand(t, slot):
        item = wid * _ITEMS_PER_WORKER + t
        r = item % _NR
        pk0a = pev[2 * r, 0:16]
        pk0b = pev[2 * r, 16:32]
        pk1a = pev[2 * r + 1, 0:16]
        pk1b = pev[2 * r + 1, 16:32]
        ovf = ov.at[slot]

        def do_token(i, c, q0, q1):
            g0 = gv[slot, i, c, 0:16] * _SCALE
            g1 = gv[slot, i, c, 16:32] * _SCALE
            if q0:
                p = i * _OW + c
                off = ((p >> 7) << 10) + (p & 127)
                plsc.store_scatter(ovf, [dlo + off], g0 + pk0a)
                plsc.store_scatter(ovf, [dhi + off], g1 + pk0b)
            if q1:
                p1 = i * _OW + c - 1
                off1 = _PLANE + ((p1 >> 7) << 10) + (p1 & 127)
                plsc.store_scatter(ovf, [dlo + off1], g0 + pk1a)
                plsc.store_scatter(ovf, [dhi + off1], g1 + pk1b)

        for i in range(8):
            do_token(i, 0, True, False)

            @plsc.parallel_loop(1, _OW, unroll=2)
            def body(c, i=i):
                do_token(i, c, True, True)

            do_token(i, _OW, False, True)

    out_descs = [None, None]
    gdescs = stage(0, 0)
    for t in range(_ITEMS_PER_WORKER):
        slot = t % 2
        for d in gdescs:
            d.wait()
        if t + 1 < _ITEMS_PER_WORKER:
            gdescs = stage(t + 1, 1 - slot)
        if out_descs[slot] is not None:
            out_descs[slot].wait()
        expand(t, slot)
        item = wid * _ITEMS_PER_WORKER + t
        out_descs[slot] = pltpu.make_async_copy(
            ov.at[slot], out_hbm.at[pl.ds(item * 2 * _PLANE, 2 * _PLANE)], osem
        )
        out_descs[slot].start()
    for d in out_descs:
        if d is not None:
            d.wait()


@functools.partial(
    pl.kernel,
    out_type=jax.ShapeDtypeStruct((_ITEMS * 2 * _PLANE,), jnp.float32),
    mesh=plsc.VectorSubcoreMesh(core_axis_name="c", subcore_axis_name="s"),
    scratch_types=[
        pltpu.VMEM((2, 8, _W), jnp.int32),
        pltpu.VMEM((2, 8, _W, _D), jnp.float32),
        pltpu.VMEM((2, 2 * _PLANE), jnp.float32),
        pltpu.VMEM((_D, _D), jnp.float32),
        pltpu.SemaphoreType.DMA,
        pltpu.SemaphoreType.DMA,
    ],
    compiler_params=pltpu.CompilerParams(
        use_tc_tiling_on_sc=False, needs_layout_passes=False
    ),
)
def _pos_emb_sc(x_hbm, pe_hbm, table_hbm, out_hbm, xv, gv, ov, pev, gsem, osem):
    _sc_body(x_hbm, pe_hbm, table_hbm, out_hbm, xv, gv, ov, pev, gsem, osem)


def kernel(x, table):
    x4 = x.reshape(_NB, 8, 16, _W)
    out1d = _pos_emb_sc(x4, _PE, table)
    out6 = out1d.reshape(_NB, _D, 4, 4, 8, 128)
    t = jnp.transpose(out6, (0, 3, 5, 1, 2, 4))
    return t.reshape(_NB, 512, _D, _D)[:, : 8 * _OW]
